# Initial kernel scaffold; baseline (speedup 1.0000x reference)
import jax
import jax.numpy as jnp
from jax import lax
from jax.experimental import pallas as pl
from jax.experimental.pallas import tpu as pltpu
from jax.experimental.pallas import tpu_sc as plsc

N = 10000
D = 128
NE = 320000
NP = 10240
EP = 327680
NC = 2
NS = 16
NW = NC * NS
KCH = EP // (NW * 128)
CB = 128
RPT = NP // NS
BLK = 1280


def _mesh():
    return plsc.VectorSubcoreMesh(core_axis_name="c", subcore_axis_name="s")


def _deg_body(srcs, dsts, degs, idx_s, idx_d, ones_v, zb, acc_o, acc_i):
    cid = lax.axis_index("c")
    sid = lax.axis_index("s")
    wid = cid * NS + sid
    pltpu.sync_copy(srcs.at[wid], idx_s)
    pltpu.sync_copy(dsts.at[wid], idx_d)
    one16 = jnp.ones((16,), jnp.float32)
    zero16 = jnp.zeros((16,), jnp.float32)

    def fill(i, carry):
        ones_v[i, :] = one16
        zb[i, :] = zero16
        return carry

    lax.fori_loop(0, CB, fill, 0)
    for j in range(RPT // CB):
        pltpu.sync_copy(zb, acc_o.at[pl.ds(sid * RPT + j * CB, CB)])
        pltpu.sync_copy(zb, acc_i.at[pl.ds(sid * RPT + j * CB, CB)])
    plsc.subcore_barrier()

    def step(k, carry):
        pltpu.sync_copy(ones_v, acc_o.at[idx_s.at[k]], add=True)
        pltpu.sync_copy(ones_v, acc_i.at[idx_d.at[k]], add=True)
        return carry

    lax.fori_loop(0, KCH, step, 0)
    plsc.subcore_barrier()
    sl = pl.ds(sid * RPT, RPT)
    pltpu.sync_copy(acc_o.at[sl], degs.at[cid, 0, sl])
    pltpu.sync_copy(acc_i.at[sl], degs.at[cid, 1, sl])


_deg_call = pl.kernel(
    _deg_body,
    out_type=jax.ShapeDtypeStruct((NC, 2, NP, 16), jnp.float32),
    mesh=_mesh(),
    scratch_types=[
        pltpu.VMEM((KCH, CB), jnp.int32),
        pltpu.VMEM((KCH, CB), jnp.int32),
        pltpu.VMEM((CB, 16), jnp.float32),
        pltpu.VMEM((CB, 16), jnp.float32),
        pltpu.VMEM_SHARED((NP, 16), jnp.float32),
        pltpu.VMEM_SHARED((NP, 16), jnp.float32),
    ],
)


def _agg_body(y, srcs, dsts, parts, idx_s, idx_d, rows0, rows1, sem0, sem1, acc):
    cid = lax.axis_index("c")
    sid = lax.axis_index("s")
    wid = cid * NS + sid
    pltpu.sync_copy(srcs.at[wid], idx_s)
    pltpu.sync_copy(dsts.at[wid], idx_d)
    zero16 = jnp.zeros((16,), jnp.float32)

    def fill(i, carry):
        for j in range(D // 16):
            rows0[i, pl.ds(j * 16, 16)] = zero16
        return carry

    lax.fori_loop(0, CB, fill, 0)
    for j in range(RPT // CB):
        pltpu.sync_copy(rows0, acc.at[pl.ds(sid * RPT + j * CB, CB)])
    plsc.subcore_barrier()

    pltpu.async_copy(y.at[idx_s.at[0]], rows0, sem0)

    def step(k2, carry):
        k = 2 * k2
        pltpu.make_async_copy(y.at[idx_s.at[k]], rows0, sem0).wait()
        pltpu.async_copy(y.at[idx_s.at[k + 1]], rows1, sem1)
        pltpu.sync_copy(rows0, acc.at[idx_d.at[k]], add=True)
        pltpu.make_async_copy(y.at[idx_s.at[k + 1]], rows1, sem1).wait()

        @pl.when(k + 2 < KCH)
        def _():
            pltpu.async_copy(y.at[idx_s.at[k + 2]], rows0, sem0)

        pltpu.sync_copy(rows1, acc.at[idx_d.at[k + 1]], add=True)
        return carry

    lax.fori_loop(0, KCH // 2, step, 0)
    plsc.subcore_barrier()
    sl = pl.ds(sid * RPT, RPT)
    pltpu.sync_copy(acc.at[sl], parts.at[cid, sl])


_agg_call = pl.kernel(
    _agg_body,
    out_type=jax.ShapeDtypeStruct((NC, NP, D), jnp.float32),
    mesh=_mesh(),
    scratch_types=[
        pltpu.VMEM((KCH, CB), jnp.int32),
        pltpu.VMEM((KCH, CB), jnp.int32),
        pltpu.VMEM((CB, D), jnp.float32),
        pltpu.VMEM((CB, D), jnp.float32),
        pltpu.SemaphoreType.DMA,
        pltpu.SemaphoreType.DMA,
        pltpu.VMEM_SHARED((NP, D), jnp.float32),
    ],
)


def _norm_from(deg0, deg1, row):
    deg = deg0[:, 0:1] + deg1[:, 0:1]
    ok = jnp.logical_and(deg > 0, row < N)
    return jnp.where(ok, lax.rsqrt(jnp.maximum(deg, 1e-12)), 0.0)


def _row_ids():
    i = pl.program_id(0)
    return i * BLK + lax.broadcasted_iota(jnp.int32, (BLK, 1), 0)


def _mm_norm_body(x_ref, w_ref, do0, do1, y_ref):
    ns = _norm_from(do0[...], do1[...], _row_ids())
    y_ref[...] = jnp.dot(x_ref[...], w_ref[...],
                         preferred_element_type=jnp.float32) * ns


def _layer_body(p0, p1, di0, di1, do0, do1, b_ref, w_ref, y_ref):
    row = _row_ids()
    nd = _norm_from(di0[...], di1[...], row)
    ns = _norm_from(do0[...], do1[...], row)
    h = (p0[...] + p1[...]) * nd + b_ref[...]
    h = jnp.where(h >= 0, h, 0.01 * h)
    y_ref[...] = jnp.dot(h, w_ref[...], preferred_element_type=jnp.float32) * ns


def _final_body(q0, q1, di0, di1, b_ref, wl_ref, bl_ref, o_ref):
    nd = _norm_from(di0[...], di1[...], _row_ids())
    h = (q0[...] + q1[...]) * nd + b_ref[...]
    h = jnp.where(h >= 0, h, 0.01 * h)
    o_ref[...] = jnp.dot(h, wl_ref[...],
                         preferred_element_type=jnp.float32) + bl_ref[...]


def _rows_spec(width):
    return pl.BlockSpec((BLK, width), lambda i: (i, 0))


def _whole_spec(shape):
    return pl.BlockSpec(shape, lambda i: (0,) * len(shape))


_mm_norm = pl.pallas_call(
    _mm_norm_body,
    grid=(NP // BLK,),
    in_specs=[_rows_spec(D), _whole_spec((D, D)), _rows_spec(16), _rows_spec(16)],
    out_specs=_rows_spec(D),
    out_shape=jax.ShapeDtypeStruct((NP, D), jnp.float32),
)

_layer = pl.pallas_call(
    _layer_body,
    grid=(NP // BLK,),
    in_specs=[_rows_spec(D), _rows_spec(D), _rows_spec(16), _rows_spec(16),
              _rows_spec(16), _rows_spec(16), _whole_spec((1, D)),
              _whole_spec((D, D))],
    out_specs=_rows_spec(D),
    out_shape=jax.ShapeDtypeStruct((NP, D), jnp.float32),
)

_final = pl.pallas_call(
    _final_body,
    grid=(NP // BLK,),
    in_specs=[_rows_spec(D), _rows_spec(D), _rows_spec(16), _rows_spec(16),
              _whole_spec((1, D)), _whole_spec((D, 8)), _whole_spec((1, 8))],
    out_specs=_rows_spec(8),
    out_shape=jax.ShapeDtypeStruct((NP, 8), jnp.float32),
)


def kernel(x, edge_index, W1, b1, W2, b2, Wl, bl):
    x_pad = jnp.pad(x, ((0, NP - N), (0, 0)))
    pad = jnp.full((EP - NE,), N, jnp.int32)
    srcs3 = jnp.concatenate([edge_index[0], pad]).reshape(NW, KCH, CB)
    dsts3 = jnp.concatenate([edge_index[1], pad]).reshape(NW, KCH, CB)

    degs = _deg_call(srcs3, dsts3)
    do0, do1 = degs[0, 0], degs[1, 0]
    di0, di1 = degs[0, 1], degs[1, 1]

    y1 = _mm_norm(x_pad, W1, do0, do1)
    p = _agg_call(y1, srcs3, dsts3)
    y2 = _layer(p[0], p[1], di0, di1, do0, do1, b1.reshape(1, D), W2)
    q = _agg_call(y2, srcs3, dsts3)

    wl_pad = jnp.zeros((D, 8), Wl.dtype).at[:, :2].set(Wl)
    bl_pad = jnp.zeros((1, 8), bl.dtype).at[0, :2].set(bl)
    outp = _final(q[0], q[1], di0, di1, b2.reshape(1, D), wl_pad, bl_pad)
    return outp[:N, :2]


# speedup vs baseline: 3.3938x; 3.3938x over previous
import jax
import jax.numpy as jnp
from jax import lax
from jax.experimental import pallas as pl
from jax.experimental.pallas import tpu as pltpu
from jax.experimental.pallas import tpu_sc as plsc

N = 10000
D = 128
NE = 320000
NP = 10240
EP = 327680
NC = 2
NS = 16
NW = NC * NS
CB = 128
KCH = EP // (NW * CB)
NPH = 2
CPP = KCH // NPH
EPP = CPP * CB
RPT = NP // NS
BLK = 1280


def _mesh():
    return plsc.VectorSubcoreMesh(core_axis_name="c", subcore_axis_name="s",
                                  num_cores=NC, num_subcores=NS)


def _deg_body(srcs, dsts, degs, idx_s, idx_d, ones_v, acc):
    cid = lax.axis_index("c")
    sid = lax.axis_index("s")
    wid = cid * NS + sid
    pltpu.sync_copy(srcs.at[wid], idx_s)
    pltpu.sync_copy(dsts.at[wid], idx_d)
    one16 = jnp.ones((16,), jnp.float32)
    zero16 = jnp.zeros((16,), jnp.float32)

    def fill_of(val16):
        def fill(i, carry):
            for j in range(D // 16):
                ones_v[i, pl.ds(j * 16, 16)] = val16
            return carry
        return fill

    sl = pl.ds(sid * RPT, RPT)
    for phase in range(2):
        lax.fori_loop(0, CB, fill_of(zero16), 0)
        for j in range(RPT // CB):
            pltpu.sync_copy(ones_v, acc.at[pl.ds(sid * RPT + j * CB, CB)])
        lax.fori_loop(0, CB, fill_of(one16), 0)
        plsc.subcore_barrier()

        idx = idx_s if phase == 0 else idx_d

        def step(k, carry):
            pltpu.sync_copy(ones_v, acc.at[idx.at[k]], add=True)
            return carry

        lax.fori_loop(0, KCH, step, 0)
        plsc.subcore_barrier()
        pltpu.sync_copy(acc.at[sl], degs.at[cid, phase, sl])
        if phase == 0:
            plsc.subcore_barrier()


_deg_call = pl.kernel(
    _deg_body,
    out_type=jax.ShapeDtypeStruct((NC, 2, NP, D), jnp.float32),
    mesh=_mesh(),
    scratch_types=[
        pltpu.VMEM((KCH, CB), jnp.int32),
        pltpu.VMEM((KCH, CB), jnp.int32),
        pltpu.VMEM((CB, D), jnp.float32),
        pltpu.VMEM_SHARED((NP, D), jnp.float32),
    ],
)


def _agg_body(y, srcs, dsts, parts, idx_s, idx_d, rows0, rows1, sem0, sem1, acc):
    cid = lax.axis_index("c")
    sid = lax.axis_index("s")
    wid = cid * NS + sid
    zero16 = jnp.zeros((16,), jnp.float32)

    def fill(i, carry):
        for j in range(D // 16):
            rows0[i, pl.ds(j * 16, 16)] = zero16
        return carry

    lax.fori_loop(0, CB, fill, 0)
    for j in range(RPT // CB):
        pltpu.sync_copy(rows0, acc.at[pl.ds(sid * RPT + j * CB, CB)])
    plsc.subcore_barrier()

    for ph in range(NPH):
        pltpu.sync_copy(srcs.at[wid, ph], idx_s)
        pltpu.sync_copy(dsts.at[wid, ph], idx_d)
        pltpu.async_copy(y.at[idx_s.at[pl.ds(0, CB)]], rows0, sem0)

        def step(k2, carry):
            k = 2 * k2
            pltpu.make_async_copy(
                y.at[idx_s.at[pl.ds(k * CB, CB)]], rows0, sem0).wait()
            pltpu.async_copy(y.at[idx_s.at[pl.ds((k + 1) * CB, CB)]], rows1, sem1)
            pltpu.sync_copy(rows0, acc.at[idx_d.at[k]], add=True)
            pltpu.make_async_copy(
                y.at[idx_s.at[pl.ds((k + 1) * CB, CB)]], rows1, sem1).wait()

            @pl.when(k + 2 < CPP)
            def _():
                pltpu.async_copy(y.at[idx_s.at[pl.ds((k + 2) * CB, CB)]],
                                 rows0, sem0)

            pltpu.sync_copy(rows1, acc.at[idx_d.at[k + 1]], add=True)
            return carry

        lax.fori_loop(0, CPP // 2, step, 0)
    plsc.subcore_barrier()
    sl = pl.ds(sid * RPT, RPT)
    pltpu.sync_copy(acc.at[sl], parts.at[cid, sl])


_agg_call = pl.kernel(
    _agg_body,
    out_type=jax.ShapeDtypeStruct((NC, NP, D), jnp.float32),
    mesh=_mesh(),
    scratch_types=[
        pltpu.VMEM((EPP,), jnp.int32),
        pltpu.VMEM((CPP, CB), jnp.int32),
        pltpu.VMEM((CB, D), jnp.float32),
        pltpu.VMEM((CB, D), jnp.float32),
        pltpu.SemaphoreType.DMA,
        pltpu.SemaphoreType.DMA,
        pltpu.VMEM_SHARED((NP, D), jnp.float32),
    ],
)


def _norm_from(deg0, deg1, row):
    deg = deg0[:, 0:1] + deg1[:, 0:1]
    ok = jnp.logical_and(deg > 0, row < N)
    return jnp.where(ok, lax.rsqrt(jnp.maximum(deg, 1e-12)), 0.0)


def _row_ids():
    i = pl.program_id(0)
    return i * BLK + lax.broadcasted_iota(jnp.int32, (BLK, 1), 0)


def _mm_norm_body(x_ref, w_ref, do0, do1, di0, di1, y_ref, ns_ref, nd_ref):
    row = _row_ids()
    ns = _norm_from(do0[...], do1[...], row)
    nd = _norm_from(di0[...], di1[...], row)
    ns_ref[...] = jnp.broadcast_to(ns, (BLK, 8))
    nd_ref[...] = jnp.broadcast_to(nd, (BLK, 8))
    y_ref[...] = jnp.dot(x_ref[...], w_ref[...],
                         preferred_element_type=jnp.float32) * ns


def _layer_body(p0, p1, ns8, nd8, b_ref, w_ref, y_ref):
    h = (p0[...] + p1[...]) * nd8[:, 0:1] + b_ref[...]
    h = jnp.where(h >= 0, h, 0.01 * h)
    y_ref[...] = jnp.dot(h, w_ref[...],
                         preferred_element_type=jnp.float32) * ns8[:, 0:1]


def _final_body(q0, q1, nd8, b_ref, wl_ref, bl_ref, o_ref):
    h = (q0[...] + q1[...]) * nd8[:, 0:1] + b_ref[...]
    h = jnp.where(h >= 0, h, 0.01 * h)
    o_ref[...] = jnp.dot(h, wl_ref[...],
                         preferred_element_type=jnp.float32) + bl_ref[...]


def _rows_spec(width):
    return pl.BlockSpec((BLK, width), lambda i: (i, 0))


def _whole_spec(shape):
    return pl.BlockSpec(shape, lambda i: (0,) * len(shape))


_mm_norm = pl.pallas_call(
    _mm_norm_body,
    grid=(NP // BLK,),
    in_specs=[_rows_spec(D), _whole_spec((D, D)), _rows_spec(D), _rows_spec(D),
              _rows_spec(D), _rows_spec(D)],
    out_specs=[_rows_spec(D), _rows_spec(8), _rows_spec(8)],
    out_shape=[jax.ShapeDtypeStruct((NP, D), jnp.float32),
               jax.ShapeDtypeStruct((NP, 8), jnp.float32),
               jax.ShapeDtypeStruct((NP, 8), jnp.float32)],
)

_layer = pl.pallas_call(
    _layer_body,
    grid=(NP // BLK,),
    in_specs=[_rows_spec(D), _rows_spec(D), _rows_spec(8), _rows_spec(8),
              _whole_spec((1, D)), _whole_spec((D, D))],
    out_specs=_rows_spec(D),
    out_shape=jax.ShapeDtypeStruct((NP, D), jnp.float32),
)

_final = pl.pallas_call(
    _final_body,
    grid=(NP // BLK,),
    in_specs=[_rows_spec(D), _rows_spec(D), _rows_spec(8),
              _whole_spec((1, D)), _whole_spec((D, 8)), _whole_spec((1, 8))],
    out_specs=_rows_spec(8),
    out_shape=jax.ShapeDtypeStruct((NP, 8), jnp.float32),
)


def kernel(x, edge_index, W1, b1, W2, b2, Wl, bl):
    x_pad = jnp.pad(x, ((0, NP - N), (0, 0)))
    pad = jnp.full((EP - NE,), N, jnp.int32)
    src_flat = jnp.concatenate([edge_index[0], pad])
    dst_flat = jnp.concatenate([edge_index[1], pad])
    srcs3 = src_flat.reshape(NW, KCH, CB)
    dsts3 = dst_flat.reshape(NW, KCH, CB)
    srcs_ph = src_flat.reshape(NW, NPH, EPP)
    dsts_ph = dst_flat.reshape(NW, NPH, CPP, CB)

    degs = _deg_call(srcs3, dsts3)
    do0, do1 = degs[0, 0], degs[1, 0]
    di0, di1 = degs[0, 1], degs[1, 1]

    y1, ns8, nd8 = _mm_norm(x_pad, W1, do0, do1, di0, di1)
    p = _agg_call(y1, srcs_ph, dsts_ph)
    y2 = _layer(p[0], p[1], ns8, nd8, b1.reshape(1, D), W2)
    q = _agg_call(y2, srcs_ph, dsts_ph)

    wl_pad = jnp.zeros((D, 8), Wl.dtype).at[:, :2].set(Wl)
    bl_pad = jnp.zeros((1, 8), bl.dtype).at[0, :2].set(bl)
    outp = _final(q[0], q[1], nd8, b2.reshape(1, D), wl_pad, bl_pad)
    return outp[:N, :2]


# speedup vs baseline: 3.5033x; 1.0323x over previous
import jax
import jax.numpy as jnp
from jax import lax
from jax.experimental import pallas as pl
from jax.experimental.pallas import tpu as pltpu
from jax.experimental.pallas import tpu_sc as plsc

N = 10000
D = 128
NE = 320000
NP = 10240
EP = 327680
NC = 2
NS = 16
NW = NC * NS
CB = 128
KCH = EP // (NW * CB)
NPH = 2
CPP = KCH // NPH
EPP = CPP * CB
RPT = NP // NS
BLK = 1280


def _mesh():
    return plsc.VectorSubcoreMesh(core_axis_name="c", subcore_axis_name="s",
                                  num_cores=NC, num_subcores=NS)


def _deg_body(srcs, dsts, degs, idx_s, idx_d, ones_v, acc):
    cid = lax.axis_index("c")
    sid = lax.axis_index("s")
    wid = cid * NS + sid
    pltpu.sync_copy(srcs.at[wid], idx_s)
    pltpu.sync_copy(dsts.at[wid], idx_d)
    one16 = jnp.ones((16,), jnp.float32)
    zero16 = jnp.zeros((16,), jnp.float32)

    def fill_of(val16):
        def fill(i, carry):
            for j in range(D // 16):
                ones_v[i, pl.ds(j * 16, 16)] = val16
            return carry
        return fill

    sl = pl.ds(sid * RPT, RPT)
    for phase in range(2):
        lax.fori_loop(0, CB, fill_of(zero16), 0)
        for j in range(RPT // CB):
            pltpu.sync_copy(ones_v, acc.at[pl.ds(sid * RPT + j * CB, CB)])
        lax.fori_loop(0, CB, fill_of(one16), 0)
        plsc.subcore_barrier()

        idx = idx_s if phase == 0 else idx_d

        def step(k, carry):
            pltpu.sync_copy(ones_v, acc.at[idx.at[k]], add=True)
            return carry

        lax.fori_loop(0, KCH, step, 0)
        plsc.subcore_barrier()
        pltpu.sync_copy(acc.at[sl], degs.at[cid, phase, sl])
        if phase == 0:
            plsc.subcore_barrier()


_deg_call = pl.kernel(
    _deg_body,
    out_type=jax.ShapeDtypeStruct((NC, 2, NP, D), jnp.float32),
    mesh=_mesh(),
    scratch_types=[
        pltpu.VMEM((KCH, CB), jnp.int32),
        pltpu.VMEM((KCH, CB), jnp.int32),
        pltpu.VMEM((CB, D), jnp.float32),
        pltpu.VMEM_SHARED((NP, D), jnp.float32),
    ],
)


QN = 4
QR = CB // QN


def _agg_body(y, srcs, dsts, parts, idx_s, idx_d, rows0, rows1,
              g00, g01, g02, g03, g10, g11, g12, g13, acc):
    cid = lax.axis_index("c")
    sid = lax.axis_index("s")
    wid = cid * NS + sid
    zero16 = jnp.zeros((16,), jnp.float32)
    gsems = ((g00, g01, g02, g03), (g10, g11, g12, g13))
    bufs = (rows0, rows1)

    def fill(i, carry):
        for j in range(D // 16):
            rows0[i, pl.ds(j * 16, 16)] = zero16
        return carry

    lax.fori_loop(0, CB, fill, 0)
    for j in range(RPT // CB):
        pltpu.sync_copy(rows0, acc.at[pl.ds(sid * RPT + j * CB, CB)])
    plsc.subcore_barrier()

    def gather_chunk(c, b):
        for q in range(QN):
            pltpu.async_copy(
                y.at[idx_s.at[pl.ds(c * CB + q * QR, QR)]],
                bufs[b].at[pl.ds(q * QR, QR)], gsems[b][q])

    def wait_chunk(c, b):
        for q in range(QN):
            pltpu.make_async_copy(
                y.at[idx_s.at[pl.ds(c * CB + q * QR, QR)]],
                bufs[b].at[pl.ds(q * QR, QR)], gsems[b][q]).wait()

    for ph in range(NPH):
        pltpu.sync_copy(srcs.at[wid, ph], idx_s)
        pltpu.sync_copy(dsts.at[wid, ph], idx_d)
        gather_chunk(0, 0)
        gather_chunk(1, 1)

        def step(k2, carry):
            for b in range(2):
                c = 2 * k2 + b
                wait_chunk(c, b)
                pltpu.sync_copy(bufs[b], acc.at[idx_d.at[c]], add=True)

                @pl.when(c + 2 < CPP)
                def _():
                    gather_chunk(c + 2, b)
            return carry

        lax.fori_loop(0, CPP // 2, step, 0)
    plsc.subcore_barrier()
    sl = pl.ds(sid * RPT, RPT)
    pltpu.sync_copy(acc.at[sl], parts.at[cid, sl])


_agg_call = pl.kernel(
    _agg_body,
    out_type=jax.ShapeDtypeStruct((NC, NP, D), jnp.float32),
    mesh=_mesh(),
    scratch_types=[
        pltpu.VMEM((EPP,), jnp.int32),
        pltpu.VMEM((CPP, CB), jnp.int32),
        pltpu.VMEM((CB, D), jnp.float32),
        pltpu.VMEM((CB, D), jnp.float32),
    ] + [pltpu.SemaphoreType.DMA] * 8 + [
        pltpu.VMEM_SHARED((NP, D), jnp.float32),
    ],
)


def _norm_from(deg0, deg1, row):
    deg = deg0[:, 0:1] + deg1[:, 0:1]
    ok = jnp.logical_and(deg > 0, row < N)
    return jnp.where(ok, lax.rsqrt(jnp.maximum(deg, 1e-12)), 0.0)


def _row_ids():
    i = pl.program_id(0)
    return i * BLK + lax.broadcasted_iota(jnp.int32, (BLK, 1), 0)


def _mm_norm_body(x_ref, w_ref, do0, do1, di0, di1, y_ref, ns_ref, nd_ref):
    row = _row_ids()
    ns = _norm_from(do0[...], do1[...], row)
    nd = _norm_from(di0[...], di1[...], row)
    ns_ref[...] = jnp.broadcast_to(ns, (BLK, 8))
    nd_ref[...] = jnp.broadcast_to(nd, (BLK, 8))
    y_ref[...] = jnp.dot(x_ref[...], w_ref[...],
                         preferred_element_type=jnp.float32) * ns


def _layer_body(p0, p1, ns8, nd8, b_ref, w_ref, y_ref):
    h = (p0[...] + p1[...]) * nd8[:, 0:1] + b_ref[...]
    h = jnp.where(h >= 0, h, 0.01 * h)
    y_ref[...] = jnp.dot(h, w_ref[...],
                         preferred_element_type=jnp.float32) * ns8[:, 0:1]


def _final_body(q0, q1, nd8, b_ref, wl_ref, bl_ref, o_ref):
    h = (q0[...] + q1[...]) * nd8[:, 0:1] + b_ref[...]
    h = jnp.where(h >= 0, h, 0.01 * h)
    o_ref[...] = jnp.dot(h, wl_ref[...],
                         preferred_element_type=jnp.float32) + bl_ref[...]


def _rows_spec(width):
    return pl.BlockSpec((BLK, width), lambda i: (i, 0))


def _whole_spec(shape):
    return pl.BlockSpec(shape, lambda i: (0,) * len(shape))


_mm_norm = pl.pallas_call(
    _mm_norm_body,
    grid=(NP // BLK,),
    in_specs=[_rows_spec(D), _whole_spec((D, D)), _rows_spec(D), _rows_spec(D),
              _rows_spec(D), _rows_spec(D)],
    out_specs=[_rows_spec(D), _rows_spec(8), _rows_spec(8)],
    out_shape=[jax.ShapeDtypeStruct((NP, D), jnp.float32),
               jax.ShapeDtypeStruct((NP, 8), jnp.float32),
               jax.ShapeDtypeStruct((NP, 8), jnp.float32)],
)

_layer = pl.pallas_call(
    _layer_body,
    grid=(NP // BLK,),
    in_specs=[_rows_spec(D), _rows_spec(D), _rows_spec(8), _rows_spec(8),
              _whole_spec((1, D)), _whole_spec((D, D))],
    out_specs=_rows_spec(D),
    out_shape=jax.ShapeDtypeStruct((NP, D), jnp.float32),
)

_final = pl.pallas_call(
    _final_body,
    grid=(NP // BLK,),
    in_specs=[_rows_spec(D), _rows_spec(D), _rows_spec(8),
              _whole_spec((1, D)), _whole_spec((D, 8)), _whole_spec((1, 8))],
    out_specs=_rows_spec(8),
    out_shape=jax.ShapeDtypeStruct((NP, 8), jnp.float32),
)


def kernel(x, edge_index, W1, b1, W2, b2, Wl, bl):
    x_pad = jnp.pad(x, ((0, NP - N), (0, 0)))
    pad = jnp.full((EP - NE,), N, jnp.int32)
    src_flat = jnp.concatenate([edge_index[0], pad])
    dst_flat = jnp.concatenate([edge_index[1], pad])
    srcs3 = src_flat.reshape(NW, KCH, CB)
    dsts3 = dst_flat.reshape(NW, KCH, CB)
    srcs_ph = src_flat.reshape(NW, NPH, EPP)
    dsts_ph = dst_flat.reshape(NW, NPH, CPP, CB)

    degs = _deg_call(srcs3, dsts3)
    do0, do1 = degs[0, 0], degs[1, 0]
    di0, di1 = degs[0, 1], degs[1, 1]

    y1, ns8, nd8 = _mm_norm(x_pad, W1, do0, do1, di0, di1)
    p = _agg_call(y1, srcs_ph, dsts_ph)
    y2 = _layer(p[0], p[1], ns8, nd8, b1.reshape(1, D), W2)
    q = _agg_call(y2, srcs_ph, dsts_ph)

    wl_pad = jnp.zeros((D, 8), Wl.dtype).at[:, :2].set(Wl)
    bl_pad = jnp.zeros((1, 8), bl.dtype).at[0, :2].set(bl)
    outp = _final(q[0], q[1], nd8, b2.reshape(1, D), wl_pad, bl_pad)
    return outp[:N, :2]


# speedup vs baseline: 3.6093x; 1.0303x over previous
import jax
import jax.numpy as jnp
from jax import lax
from jax.experimental import pallas as pl
from jax.experimental.pallas import tpu as pltpu
from jax.experimental.pallas import tpu_sc as plsc

N = 10000
D = 128
NE = 320000
NP = 10240
EP = 327680
NC = 2
NS = 16
NW = NC * NS
CB = 128
KCH = EP // (NW * CB)
NPH = 4
CPF = 31
CPS = 9
FAST_CORE = 0
EF = NS * NPH * CPF * CB
ES = NS * NPH * CPS * CB
RPT = NP // NS
BLK = 1280


def _mesh():
    return plsc.VectorSubcoreMesh(core_axis_name="c", subcore_axis_name="s",
                                  num_cores=NC, num_subcores=NS)


def _deg_body(srcs, dsts, degs, idx_s, idx_d, ones_v, acc):
    cid = lax.axis_index("c")
    sid = lax.axis_index("s")
    wid = cid * NS + sid
    pltpu.sync_copy(srcs.at[wid], idx_s)
    pltpu.sync_copy(dsts.at[wid], idx_d)
    one16 = jnp.ones((16,), jnp.float32)
    zero16 = jnp.zeros((16,), jnp.float32)

    def fill_of(val16):
        def fill(i, carry):
            for j in range(D // 16):
                ones_v[i, pl.ds(j * 16, 16)] = val16
            return carry
        return fill

    sl = pl.ds(sid * RPT, RPT)
    for phase in range(2):
        lax.fori_loop(0, CB, fill_of(zero16), 0)
        for j in range(RPT // CB):
            pltpu.sync_copy(ones_v, acc.at[pl.ds(sid * RPT + j * CB, CB)])
        lax.fori_loop(0, CB, fill_of(one16), 0)
        plsc.subcore_barrier()

        idx = idx_s if phase == 0 else idx_d

        def step(k, carry):
            pltpu.sync_copy(ones_v, acc.at[idx.at[k]], add=True)
            return carry

        lax.fori_loop(0, KCH, step, 0)
        plsc.subcore_barrier()
        pltpu.sync_copy(acc.at[sl], degs.at[cid, phase, sl])
        if phase == 0:
            plsc.subcore_barrier()


_deg_call = pl.kernel(
    _deg_body,
    out_type=jax.ShapeDtypeStruct((NC, 2, NP, D), jnp.float32),
    mesh=_mesh(),
    scratch_types=[
        pltpu.VMEM((KCH, CB), jnp.int32),
        pltpu.VMEM((KCH, CB), jnp.int32),
        pltpu.VMEM((CB, D), jnp.float32),
        pltpu.VMEM_SHARED((NP, D), jnp.float32),
    ],
)


QN = 4
QR = CB // QN


def _agg_body(y, srcs_f, dsts_f, srcs_s, dsts_s, parts, idx_s, idx_d,
              rows0, rows1, g00, g01, g02, g03, g10, g11, g12, g13, acc):
    cid = lax.axis_index("c")
    sid = lax.axis_index("s")
    zero16 = jnp.zeros((16,), jnp.float32)
    gsems = ((g00, g01, g02, g03), (g10, g11, g12, g13))
    bufs = (rows0, rows1)

    def fill(i, carry):
        for j in range(D // 16):
            rows0[i, pl.ds(j * 16, 16)] = zero16
        return carry

    lax.fori_loop(0, CB, fill, 0)
    for j in range(RPT // CB):
        pltpu.sync_copy(rows0, acc.at[pl.ds(sid * RPT + j * CB, CB)])
    plsc.subcore_barrier()

    def gather_chunk(c, b):
        for q in range(QN):
            pltpu.async_copy(
                y.at[idx_s.at[pl.ds(c * CB + q * QR, QR)]],
                bufs[b].at[pl.ds(q * QR, QR)], gsems[b][q])

    def wait_chunk(c, b):
        for q in range(QN):
            pltpu.make_async_copy(
                y.at[idx_s.at[pl.ds(c * CB + q * QR, QR)]],
                bufs[b].at[pl.ds(q * QR, QR)], gsems[b][q]).wait()

    def run_side(srcs, dsts, ncH):
        for ph in range(NPH):
            pltpu.sync_copy(srcs.at[sid, ph], idx_s.at[pl.ds(0, ncH * CB)])
            pltpu.sync_copy(dsts.at[sid, ph], idx_d.at[pl.ds(0, ncH)])
            gather_chunk(0, 0)
            gather_chunk(1, 1)

            def step(k2, carry):
                for b in range(2):
                    c = 2 * k2 + b

                    @pl.when(c < ncH)
                    def _():
                        wait_chunk(c, b)
                        pltpu.sync_copy(bufs[b], acc.at[idx_d.at[c]], add=True)

                        @pl.when(c + 2 < ncH)
                        def _():
                            gather_chunk(c + 2, b)
                return carry

            lax.fori_loop(0, (ncH + 1) // 2, step, 0)

    @pl.when(cid == FAST_CORE)
    def _():
        run_side(srcs_f, dsts_f, CPF)

    @pl.when(cid != FAST_CORE)
    def _():
        run_side(srcs_s, dsts_s, CPS)

    plsc.subcore_barrier()
    sl = pl.ds(sid * RPT, RPT)
    pltpu.sync_copy(acc.at[sl], parts.at[cid, sl])


_agg_call = pl.kernel(
    _agg_body,
    out_type=jax.ShapeDtypeStruct((NC, NP, D), jnp.float32),
    mesh=_mesh(),
    scratch_types=[
        pltpu.VMEM((CPF * CB,), jnp.int32),
        pltpu.VMEM((CPF, CB), jnp.int32),
        pltpu.VMEM((CB, D), jnp.float32),
        pltpu.VMEM((CB, D), jnp.float32),
    ] + [pltpu.SemaphoreType.DMA] * 8 + [
        pltpu.VMEM_SHARED((NP, D), jnp.float32),
    ],
)


def _norm_from(deg0, deg1, row):
    deg = deg0[:, 0:1] + deg1[:, 0:1]
    ok = jnp.logical_and(deg > 0, row < N)
    return jnp.where(ok, lax.rsqrt(jnp.maximum(deg, 1e-12)), 0.0)


def _row_ids():
    i = pl.program_id(0)
    return i * BLK + lax.broadcasted_iota(jnp.int32, (BLK, 1), 0)


def _mm_norm_body(x_ref, w_ref, do0, do1, di0, di1, y_ref, ns_ref, nd_ref):
    row = _row_ids()
    ns = _norm_from(do0[...], do1[...], row)
    nd = _norm_from(di0[...], di1[...], row)
    ns_ref[...] = jnp.broadcast_to(ns, (BLK, 8))
    nd_ref[...] = jnp.broadcast_to(nd, (BLK, 8))
    y_ref[...] = jnp.dot(x_ref[...], w_ref[...],
                         preferred_element_type=jnp.float32) * ns


def _layer_body(p0, p1, ns8, nd8, b_ref, w_ref, y_ref):
    h = (p0[...] + p1[...]) * nd8[:, 0:1] + b_ref[...]
    h = jnp.where(h >= 0, h, 0.01 * h)
    y_ref[...] = jnp.dot(h, w_ref[...],
                         preferred_element_type=jnp.float32) * ns8[:, 0:1]


def _final_body(q0, q1, nd8, b_ref, wl_ref, bl_ref, o_ref):
    h = (q0[...] + q1[...]) * nd8[:, 0:1] + b_ref[...]
    h = jnp.where(h >= 0, h, 0.01 * h)
    o_ref[...] = jnp.dot(h, wl_ref[...],
                         preferred_element_type=jnp.float32) + bl_ref[...]


def _rows_spec(width):
    return pl.BlockSpec((BLK, width), lambda i: (i, 0))


def _whole_spec(shape):
    return pl.BlockSpec(shape, lambda i: (0,) * len(shape))


_mm_norm = pl.pallas_call(
    _mm_norm_body,
    grid=(NP // BLK,),
    in_specs=[_rows_spec(D), _whole_spec((D, D)), _rows_spec(D), _rows_spec(D),
              _rows_spec(D), _rows_spec(D)],
    out_specs=[_rows_spec(D), _rows_spec(8), _rows_spec(8)],
    out_shape=[jax.ShapeDtypeStruct((NP, D), jnp.float32),
               jax.ShapeDtypeStruct((NP, 8), jnp.float32),
               jax.ShapeDtypeStruct((NP, 8), jnp.float32)],
)

_layer = pl.pallas_call(
    _layer_body,
    grid=(NP // BLK,),
    in_specs=[_rows_spec(D), _rows_spec(D), _rows_spec(8), _rows_spec(8),
              _whole_spec((1, D)), _whole_spec((D, D))],
    out_specs=_rows_spec(D),
    out_shape=jax.ShapeDtypeStruct((NP, D), jnp.float32),
)

_final = pl.pallas_call(
    _final_body,
    grid=(NP // BLK,),
    in_specs=[_rows_spec(D), _rows_spec(D), _rows_spec(8),
              _whole_spec((1, D)), _whole_spec((D, 8)), _whole_spec((1, 8))],
    out_specs=_rows_spec(8),
    out_shape=jax.ShapeDtypeStruct((NP, 8), jnp.float32),
)


def kernel(x, edge_index, W1, b1, W2, b2, Wl, bl):
    x_pad = jnp.pad(x, ((0, NP - N), (0, 0)))
    pad = jnp.full((EP - NE,), N, jnp.int32)
    src_flat = jnp.concatenate([edge_index[0], pad])
    dst_flat = jnp.concatenate([edge_index[1], pad])
    srcs3 = src_flat.reshape(NW, KCH, CB)
    dsts3 = dst_flat.reshape(NW, KCH, CB)
    srcs_f = src_flat[:EF].reshape(NS, NPH, CPF * CB)
    dsts_f = dst_flat[:EF].reshape(NS, NPH, CPF, CB)
    srcs_s = src_flat[EF:].reshape(NS, NPH, CPS * CB)
    dsts_s = dst_flat[EF:].reshape(NS, NPH, CPS, CB)

    degs = _deg_call(srcs3, dsts3)
    do0, do1 = degs[0, 0], degs[1, 0]
    di0, di1 = degs[0, 1], degs[1, 1]

    y1, ns8, nd8 = _mm_norm(x_pad, W1, do0, do1, di0, di1)
    p = _agg_call(y1, srcs_f, dsts_f, srcs_s, dsts_s)
    y2 = _layer(p[0], p[1], ns8, nd8, b1.reshape(1, D), W2)
    q = _agg_call(y2, srcs_f, dsts_f, srcs_s, dsts_s)

    wl_pad = jnp.zeros((D, 8), Wl.dtype).at[:, :2].set(Wl)
    bl_pad = jnp.zeros((1, 8), bl.dtype).at[0, :2].set(bl)
    outp = _final(q[0], q[1], nd8, b2.reshape(1, D), wl_pad, bl_pad)
    return outp[:N, :2]


# speedup vs baseline: 3.6132x; 1.0011x over previous
import jax
import jax.numpy as jnp
from jax import lax
from jax.experimental import pallas as pl
from jax.experimental.pallas import tpu as pltpu
from jax.experimental.pallas import tpu_sc as plsc

N = 10000
D = 128
NE = 320000
NP = 10240
EP = 327680
NC = 2
NS = 16
NW = NC * NS
CB = 128
KCH = EP // (NW * CB)
NPH = 4
CPF = 31
CPS = 9
FAST_CORE = 1
EF = NS * NPH * CPF * CB
ES = NS * NPH * CPS * CB
RPT = NP // NS
BLK = 1280


def _mesh():
    return plsc.VectorSubcoreMesh(core_axis_name="c", subcore_axis_name="s",
                                  num_cores=NC, num_subcores=NS)


def _deg_body(srcs, dsts, degs, idx_s, idx_d, ones_v, acc):
    cid = lax.axis_index("c")
    sid = lax.axis_index("s")
    wid = cid * NS + sid
    pltpu.sync_copy(srcs.at[wid], idx_s)
    pltpu.sync_copy(dsts.at[wid], idx_d)
    one16 = jnp.ones((16,), jnp.float32)
    zero16 = jnp.zeros((16,), jnp.float32)

    def fill_of(val16):
        def fill(i, carry):
            for j in range(D // 16):
                ones_v[i, pl.ds(j * 16, 16)] = val16
            return carry
        return fill

    sl = pl.ds(sid * RPT, RPT)
    for phase in range(2):
        lax.fori_loop(0, CB, fill_of(zero16), 0)
        for j in range(RPT // CB):
            pltpu.sync_copy(ones_v, acc.at[pl.ds(sid * RPT + j * CB, CB)])
        lax.fori_loop(0, CB, fill_of(one16), 0)
        plsc.subcore_barrier()

        idx = idx_s if phase == 0 else idx_d

        def step(k, carry):
            pltpu.sync_copy(ones_v, acc.at[idx.at[k]], add=True)
            return carry

        lax.fori_loop(0, KCH, step, 0)
        plsc.subcore_barrier()
        pltpu.sync_copy(acc.at[sl], degs.at[cid, phase, sl])
        if phase == 0:
            plsc.subcore_barrier()


_deg_call = pl.kernel(
    _deg_body,
    out_type=jax.ShapeDtypeStruct((NC, 2, NP, D), jnp.float32),
    mesh=_mesh(),
    scratch_types=[
        pltpu.VMEM((KCH, CB), jnp.int32),
        pltpu.VMEM((KCH, CB), jnp.int32),
        pltpu.VMEM((CB, D), jnp.float32),
        pltpu.VMEM_SHARED((NP, D), jnp.float32),
    ],
)


QN = 4
QR = CB // QN


def _agg_body(y, srcs_f, dsts_f, srcs_s, dsts_s, parts, idx_s, idx_d,
              rows0, rows1, g00, g01, g02, g03, g10, g11, g12, g13, acc):
    cid = lax.axis_index("c")
    sid = lax.axis_index("s")
    zero16 = jnp.zeros((16,), jnp.float32)
    gsems = ((g00, g01, g02, g03), (g10, g11, g12, g13))
    bufs = (rows0, rows1)

    def fill(i, carry):
        for j in range(D // 16):
            rows0[i, pl.ds(j * 16, 16)] = zero16
        return carry

    lax.fori_loop(0, CB, fill, 0)
    for j in range(RPT // CB):
        pltpu.sync_copy(rows0, acc.at[pl.ds(sid * RPT + j * CB, CB)])
    plsc.subcore_barrier()

    def gather_chunk(c, b):
        for q in range(QN):
            pltpu.async_copy(
                y.at[idx_s.at[pl.ds(c * CB + q * QR, QR)]],
                bufs[b].at[pl.ds(q * QR, QR)], gsems[b][q])

    def wait_chunk(c, b):
        for q in range(QN):
            pltpu.make_async_copy(
                y.at[idx_s.at[pl.ds(c * CB + q * QR, QR)]],
                bufs[b].at[pl.ds(q * QR, QR)], gsems[b][q]).wait()

    def run_side(srcs, dsts, ncH):
        for ph in range(NPH):
            pltpu.sync_copy(srcs.at[sid, ph], idx_s.at[pl.ds(0, ncH * CB)])
            pltpu.sync_copy(dsts.at[sid, ph], idx_d.at[pl.ds(0, ncH)])
            gather_chunk(0, 0)
            gather_chunk(1, 1)

            def step(k2, carry):
                for b in range(2):
                    c = 2 * k2 + b

                    @pl.when(c < ncH)
                    def _():
                        wait_chunk(c, b)
                        pltpu.sync_copy(bufs[b], acc.at[idx_d.at[c]], add=True)

                        @pl.when(c + 2 < ncH)
                        def _():
                            gather_chunk(c + 2, b)
                return carry

            lax.fori_loop(0, (ncH + 1) // 2, step, 0)

    @pl.when(cid == FAST_CORE)
    def _():
        run_side(srcs_f, dsts_f, CPF)

    @pl.when(cid != FAST_CORE)
    def _():
        run_side(srcs_s, dsts_s, CPS)

    plsc.subcore_barrier()
    sl = pl.ds(sid * RPT, RPT)
    pltpu.sync_copy(acc.at[sl], parts.at[cid, sl])


_agg_call = pl.kernel(
    _agg_body,
    out_type=jax.ShapeDtypeStruct((NC, NP, D), jnp.float32),
    mesh=_mesh(),
    scratch_types=[
        pltpu.VMEM((CPF * CB,), jnp.int32),
        pltpu.VMEM((CPF, CB), jnp.int32),
        pltpu.VMEM((CB, D), jnp.float32),
        pltpu.VMEM((CB, D), jnp.float32),
    ] + [pltpu.SemaphoreType.DMA] * 8 + [
        pltpu.VMEM_SHARED((NP, D), jnp.float32),
    ],
)


def _norm_from(deg0, deg1, row):
    deg = deg0[:, 0:1] + deg1[:, 0:1]
    ok = jnp.logical_and(deg > 0, row < N)
    return jnp.where(ok, lax.rsqrt(jnp.maximum(deg, 1e-12)), 0.0)


def _row_ids():
    i = pl.program_id(0)
    return i * BLK + lax.broadcasted_iota(jnp.int32, (BLK, 1), 0)


def _mm_norm_body(x_ref, w_ref, do0, do1, di0, di1, y_ref, ns_ref, nd_ref):
    row = _row_ids()
    ns = _norm_from(do0[...], do1[...], row)
    nd = _norm_from(di0[...], di1[...], row)
    ns_ref[...] = jnp.broadcast_to(ns, (BLK, 8))
    nd_ref[...] = jnp.broadcast_to(nd, (BLK, 8))
    y_ref[...] = jnp.dot(x_ref[...], w_ref[...],
                         preferred_element_type=jnp.float32) * ns


def _layer_body(p0, p1, ns8, nd8, b_ref, w_ref, y_ref):
    h = (p0[...] + p1[...]) * nd8[:, 0:1] + b_ref[...]
    h = jnp.where(h >= 0, h, 0.01 * h)
    y_ref[...] = jnp.dot(h, w_ref[...],
                         preferred_element_type=jnp.float32) * ns8[:, 0:1]


def _final_body(q0, q1, nd8, b_ref, wl_ref, bl_ref, o_ref):
    h = (q0[...] + q1[...]) * nd8[:, 0:1] + b_ref[...]
    h = jnp.where(h >= 0, h, 0.01 * h)
    o_ref[...] = jnp.dot(h, wl_ref[...],
                         preferred_element_type=jnp.float32) + bl_ref[...]


def _rows_spec(width):
    return pl.BlockSpec((BLK, width), lambda i: (i, 0))


def _whole_spec(shape):
    return pl.BlockSpec(shape, lambda i: (0,) * len(shape))


_mm_norm = pl.pallas_call(
    _mm_norm_body,
    grid=(NP // BLK,),
    in_specs=[_rows_spec(D), _whole_spec((D, D)), _rows_spec(D), _rows_spec(D),
              _rows_spec(D), _rows_spec(D)],
    out_specs=[_rows_spec(D), _rows_spec(8), _rows_spec(8)],
    out_shape=[jax.ShapeDtypeStruct((NP, D), jnp.float32),
               jax.ShapeDtypeStruct((NP, 8), jnp.float32),
               jax.ShapeDtypeStruct((NP, 8), jnp.float32)],
)

_layer = pl.pallas_call(
    _layer_body,
    grid=(NP // BLK,),
    in_specs=[_rows_spec(D), _rows_spec(D), _rows_spec(8), _rows_spec(8),
              _whole_spec((1, D)), _whole_spec((D, D))],
    out_specs=_rows_spec(D),
    out_shape=jax.ShapeDtypeStruct((NP, D), jnp.float32),
)

_final = pl.pallas_call(
    _final_body,
    grid=(NP // BLK,),
    in_specs=[_rows_spec(D), _rows_spec(D), _rows_spec(8),
              _whole_spec((1, D)), _whole_spec((D, 8)), _whole_spec((1, 8))],
    out_specs=_rows_spec(8),
    out_shape=jax.ShapeDtypeStruct((NP, 8), jnp.float32),
)


def kernel(x, edge_index, W1, b1, W2, b2, Wl, bl):
    x_pad = jnp.pad(x, ((0, NP - N), (0, 0)))
    pad = jnp.full((EP - NE,), N, jnp.int32)
    src_flat = jnp.concatenate([edge_index[0], pad])
    dst_flat = jnp.concatenate([edge_index[1], pad])
    srcs3 = src_flat.reshape(NW, KCH, CB)
    dsts3 = dst_flat.reshape(NW, KCH, CB)
    srcs_f = src_flat[:EF].reshape(NS, NPH, CPF * CB)
    dsts_f = dst_flat[:EF].reshape(NS, NPH, CPF, CB)
    srcs_s = src_flat[EF:].reshape(NS, NPH, CPS * CB)
    dsts_s = dst_flat[EF:].reshape(NS, NPH, CPS, CB)

    degs = _deg_call(srcs3, dsts3)
    do0, do1 = degs[0, 0], degs[1, 0]
    di0, di1 = degs[0, 1], degs[1, 1]

    y1, ns8, nd8 = _mm_norm(x_pad, W1, do0, do1, di0, di1)
    p = _agg_call(y1, srcs_f, dsts_f, srcs_s, dsts_s)
    y2 = _layer(p[0], p[1], ns8, nd8, b1.reshape(1, D), W2)
    q = _agg_call(y2, srcs_f, dsts_f, srcs_s, dsts_s)

    wl_pad = jnp.zeros((D, 8), Wl.dtype).at[:, :2].set(Wl)
    bl_pad = jnp.zeros((1, 8), bl.dtype).at[0, :2].set(bl)
    outp = _final(q[0], q[1], nd8, b2.reshape(1, D), wl_pad, bl_pad)
    return outp[:N, :2]
